# baseline (speedup 1.0000x reference)
import functools

import jax
import jax.numpy as jnp
from jax import lax
from jax.experimental import pallas as pl
from jax.experimental.pallas import tpu as pltpu
from jax.experimental.pallas import tpu_sc as plsc

NUM_CORES = 2
NUM_SUBCORES = 16
LANES = 16
NW = NUM_CORES * NUM_SUBCORES

BATCH = 16384
EMB = 32
B_W = BATCH // NW
IDX_CHUNK = 128
N_CHUNKS = B_W // IDX_CHUNK
VECS = B_W // LANES


def _gmf_body(users_h, items_h, ut_h, it_h, w_h, b_h, out_h,
              uidx_v, iidx_v, ubuf_v, ibuf_v, outv_v, w_v, b_v, sem):
    wid = lax.axis_index("s") * NUM_CORES + lax.axis_index("c")

    pltpu.sync_copy(users_h.at[wid], uidx_v)
    pltpu.sync_copy(items_h.at[wid], iidx_v)
    pltpu.sync_copy(w_h, w_v)
    pltpu.sync_copy(b_h, b_v)

    copies = []
    for c in range(EMB):
        for k in range(N_CHUNKS):
            dst = pl.ds(c * B_W + k * IDX_CHUNK, IDX_CHUNK)
            copies.append(
                pltpu.async_copy(ut_h.at[c].at[uidx_v.at[k]], ubuf_v.at[dst], sem))
            copies.append(
                pltpu.async_copy(it_h.at[c].at[iidx_v.at[k]], ibuf_v.at[dst], sem))
    for cp in copies:
        cp.wait()

    b_vec = b_v[...]

    def vec_group(v, carry):
        base = v * LANES
        acc0 = b_vec
        acc1 = jnp.zeros((LANES,), jnp.float32)
        acc2 = jnp.zeros((LANES,), jnp.float32)
        acc3 = jnp.zeros((LANES,), jnp.float32)
        accs = [acc0, acc1, acc2, acc3]
        for c in range(EMB):
            u = ubuf_v[pl.ds(c * B_W + base, LANES)]
            i = ibuf_v[pl.ds(c * B_W + base, LANES)]
            w = w_v[pl.ds(c * LANES, LANES)]
            accs[c % 4] = accs[c % 4] + (u * i) * w
        z = (accs[0] + accs[1]) + (accs[2] + accs[3])
        outv_v[pl.ds(base, LANES)] = 1.0 / (1.0 + jnp.exp(-z))
        return carry

    lax.fori_loop(0, VECS, vec_group, 0)
    pltpu.sync_copy(outv_v, out_h.at[wid])


@functools.partial(jax.jit, static_argnames=("interpret",))
def _gmf(users, items, ut_t, it_t, w_rep, b_vec, interpret=False):
    run = pl.kernel(
        _gmf_body,
        out_type=jax.ShapeDtypeStruct((NW, B_W), jnp.float32),
        mesh=plsc.VectorSubcoreMesh(core_axis_name="c", subcore_axis_name="s",
                                    num_cores=NUM_CORES, num_subcores=NUM_SUBCORES),
        scratch_types=[
            pltpu.VMEM((N_CHUNKS, IDX_CHUNK), jnp.int32),
            pltpu.VMEM((N_CHUNKS, IDX_CHUNK), jnp.int32),
            pltpu.VMEM((EMB * B_W,), jnp.float32),
            pltpu.VMEM((EMB * B_W,), jnp.float32),
            pltpu.VMEM((B_W,), jnp.float32),
            pltpu.VMEM((EMB * LANES,), jnp.float32),
            pltpu.VMEM((LANES,), jnp.float32),
            pltpu.SemaphoreType.DMA,
        ],
        compiler_params=pltpu.CompilerParams(needs_layout_passes=False,
                                             use_tc_tiling_on_sc=False),
        interpret=interpret,
    )
    return run(users, items, ut_t, it_t, w_rep, b_vec)


def kernel(users, items, user_table, item_table, W, b):
    users3 = users.astype(jnp.int32).reshape(NW, N_CHUNKS, IDX_CHUNK)
    items3 = items.astype(jnp.int32).reshape(NW, N_CHUNKS, IDX_CHUNK)
    ut_t = user_table.T
    it_t = item_table.T
    w_rep = jnp.broadcast_to(W.reshape(EMB, 1).astype(jnp.float32),
                             (EMB, LANES)).reshape(EMB * LANES)
    b_vec = jnp.broadcast_to(b.astype(jnp.float32), (LANES,))
    out = _gmf(users3, items3, ut_t, it_t, w_rep, b_vec)
    return out.reshape(BATCH, 1)


# speedup vs baseline: 24.2005x; 24.2005x over previous
import functools

import jax
import jax.numpy as jnp
from jax import lax
from jax.experimental import pallas as pl
from jax.experimental.pallas import tpu as pltpu
from jax.experimental.pallas import tpu_sc as plsc

NUM_CORES = 2
NUM_SUBCORES = 16
LANES = 16
NW = NUM_CORES * NUM_SUBCORES

BATCH = 16384
EMB = 32
B_W = BATCH // NW
GROUPS = B_W // LANES
BLK = 128
NSLOTS = 8


def _gmf_body(users_h, items_h, ut_h, it_h, w_h, b_h, out_h,
              uidx_v, iidx_v, ublk_v, iblk_v, tsc_v, outv_v, w_v, b_v,
              *sems):
    wid = lax.axis_index("s") * NUM_CORES + lax.axis_index("c")

    pltpu.sync_copy(users_h.at[wid], uidx_v)
    pltpu.sync_copy(items_h.at[wid], iidx_v)
    pltpu.sync_copy(w_h, w_v)
    pltpu.sync_copy(b_h, b_v)

    w0 = w_v[pl.ds(0, LANES)]
    w1 = w_v[pl.ds(LANES, LANES)]
    b_vec = b_v[...]
    iota = lax.iota(jnp.int32, LANES)
    zeros16 = jnp.zeros((LANES,), jnp.int32)

    def fire(slot, r_u, r_i):
        q_u = pl.multiple_of((r_u // BLK) * BLK, BLK)
        q_i = pl.multiple_of((r_i // BLK) * BLK, BLK)
        pltpu.async_copy(ut_h.at[:, pl.ds(q_u, BLK)], ublk_v.at[slot], sems[slot])
        pltpu.async_copy(it_h.at[:, pl.ds(q_i, BLK)], iblk_v.at[slot], sems[slot])

    def drain(slot):
        pltpu.make_async_copy(ut_h.at[:, pl.ds(0, BLK)], ublk_v.at[slot],
                              sems[slot]).wait()
        pltpu.make_async_copy(it_h.at[:, pl.ds(0, BLK)], iblk_v.at[slot],
                              sems[slot]).wait()

    rv_u0 = uidx_v[pl.ds(0, LANES)]
    rv_i0 = iidx_v[pl.ds(0, LANES)]
    for j in range(NSLOTS):
        fire(j, rv_u0[j], rv_i0[j])

    def group(g, carry):
        base = g * LANES
        rv_u = uidx_v[pl.ds(base, LANES)]
        rv_i = iidx_v[pl.ds(base, LANES)]
        g_next = jnp.minimum(g + 1, GROUPS - 1)
        rv_un = uidx_v[pl.ds(g_next * LANES, LANES)]
        rv_in = iidx_v[pl.ds(g_next * LANES, LANES)]

        for j in range(LANES):
            slot = j % NSLOTS
            drain(slot)
            m_u = zeros16 + (rv_u[j] % BLK)
            m_i = zeros16 + (rv_i[j] % BLK)
            u0 = plsc.load_gather(ublk_v.at[slot], [iota, m_u])
            u1 = plsc.load_gather(ublk_v.at[slot], [iota + LANES, m_u])
            i0 = plsc.load_gather(iblk_v.at[slot], [iota, m_i])
            i1 = plsc.load_gather(iblk_v.at[slot], [iota + LANES, m_i])
            tsc_v[j, pl.ds(0, LANES)] = (u0 * i0) * w0 + (u1 * i1) * w1
            if j + NSLOTS < LANES:
                fire(slot, rv_u[j + NSLOTS], rv_i[j + NSLOTS])
            else:
                @pl.when(g < GROUPS - 1)
                def _():
                    fire(slot, rv_un[j + NSLOTS - LANES], rv_in[j + NSLOTS - LANES])

        acc = b_vec
        for c in range(LANES):
            col = jnp.full((LANES,), c, jnp.int32)
            acc = acc + plsc.load_gather(tsc_v, [iota, col])
        outv_v[pl.ds(base, LANES)] = 1.0 / (1.0 + jnp.exp(-acc))
        return carry

    lax.fori_loop(0, GROUPS, group, 0)
    pltpu.sync_copy(outv_v, out_h.at[wid])


@functools.partial(jax.jit, static_argnames=("interpret",))
def _gmf(users, items, ut_t, it_t, w_flat, b_vec, interpret=False):
    run = pl.kernel(
        _gmf_body,
        out_type=jax.ShapeDtypeStruct((NW, B_W), jnp.float32),
        mesh=plsc.VectorSubcoreMesh(core_axis_name="c", subcore_axis_name="s",
                                    num_cores=NUM_CORES, num_subcores=NUM_SUBCORES),
        scratch_types=[
            pltpu.VMEM((B_W,), jnp.int32),
            pltpu.VMEM((B_W,), jnp.int32),
            pltpu.VMEM((NSLOTS, EMB, BLK), jnp.float32),
            pltpu.VMEM((NSLOTS, EMB, BLK), jnp.float32),
            pltpu.VMEM((LANES, BLK), jnp.float32),
            pltpu.VMEM((B_W,), jnp.float32),
            pltpu.VMEM((EMB,), jnp.float32),
            pltpu.VMEM((LANES,), jnp.float32),
        ] + [pltpu.SemaphoreType.DMA] * NSLOTS,
        compiler_params=pltpu.CompilerParams(needs_layout_passes=False,
                                             use_tc_tiling_on_sc=True),
        interpret=interpret,
    )
    return run(users, items, ut_t, it_t, w_flat, b_vec)


def kernel(users, items, user_table, item_table, W, b):
    users2 = users.astype(jnp.int32).reshape(NW, B_W)
    items2 = items.astype(jnp.int32).reshape(NW, B_W)
    ut_t = user_table.T
    it_t = item_table.T
    w_flat = W.reshape(EMB).astype(jnp.float32)
    b_vec = jnp.broadcast_to(b.astype(jnp.float32), (LANES,))
    out = _gmf(users2, items2, ut_t, it_t, w_flat, b_vec)
    return out.reshape(BATCH, 1)
